# baseline (speedup 1.0000x reference)
import functools
import jax
import jax.numpy as jnp
from jax import lax
from jax.experimental import pallas as pl
from jax.experimental.pallas import tpu as pltpu, tpu_sc as plsc

VOCAB = 32768
ROWS = 2048
NC = 2
NS = 16
NW = NC * NS
RPW = ROWS // NW

_mesh = plsc.VectorSubcoreMesh(core_axis_name="c", subcore_axis_name="s")


@functools.partial(
    pl.kernel,
    out_type=jax.ShapeDtypeStruct((ROWS * VOCAB,), jnp.float32),
    mesh=_mesh,
    scratch_types=[
        pltpu.VMEM((VOCAB,), jnp.float32),
        pltpu.VMEM((RPW,), jnp.int32),
        pltpu.VMEM((RPW,), jnp.int32),
        pltpu.VMEM((RPW,), jnp.float32),
        pltpu.SemaphoreType.DMA,
    ],
)
def _sc_onehot(ids_hbm, out_hbm, buf, ids_v, idx_v, zero_v, sem):
    wid = lax.axis_index("s") * NC + lax.axis_index("c")
    base_row = wid * RPW

    neg = jnp.full((16,), -1000.0, dtype=jnp.float32)

    def fill_body(j, _):
        for k in range(8):
            buf[pl.ds(j * 128 + k * 16, 16)] = neg
        return 0

    lax.fori_loop(0, VOCAB // 128, fill_body, 0)

    pltpu.sync_copy(ids_hbm.at[pl.ds(base_row, RPW)], ids_v)

    def fire_body(r, _):
        pltpu.async_copy(
            buf, out_hbm.at[pl.ds((base_row + r) * VOCAB, VOCAB)], sem
        )
        return 0

    lax.fori_loop(0, RPW, fire_body, 0)

    lane = lax.iota(jnp.int32, 16)
    zeros16 = jnp.zeros((16,), dtype=jnp.float32)
    for i in range(RPW // 16):
        ids16 = ids_v[pl.ds(i * 16, 16)]
        rows16 = base_row + i * 16 + lane
        flat16 = rows16 * VOCAB + ((ids16 + 1) & (VOCAB - 1))
        idx_v[pl.ds(i * 16, 16)] = flat16
        zero_v[pl.ds(i * 16, 16)] = zeros16

    def drain_body(r, _):
        pltpu.make_async_copy(
            buf, out_hbm.at[pl.ds((base_row + r) * VOCAB, VOCAB)], sem
        ).wait()
        return 0

    lax.fori_loop(0, RPW, drain_body, 0)

    pltpu.sync_copy(zero_v, out_hbm.at[idx_v])


def kernel(input_ids, anchor):
    batch, seq_len = input_ids.shape
    ids_flat = input_ids.reshape(batch * seq_len).astype(jnp.int32)
    out = _sc_onehot(ids_flat)
    return out.reshape(batch, seq_len, VOCAB).astype(anchor.dtype)
